# Initial kernel scaffold; baseline (speedup 1.0000x reference)
import functools

import jax
import jax.numpy as jnp
from jax import lax
from jax.experimental import pallas as pl
from jax.experimental.pallas import tpu as pltpu
from jax.experimental.pallas import tpu_sc as plsc

_VOCAB2 = 1002
_NPOS = 24
_EMBED = 128
_NC, _NS = 2, 16
_NW = _NC * _NS
_N = 1024 * 200
_CHUNK = 128
_ROWS_W = _N // _NW
_NCH = _ROWS_W // _CHUNK
_IDXROWS_W = _ROWS_W // 128


def _build_comb(W_word, W_pos):
    def body(w_ref, p_ref, o_ref):
        o_ref[...] = w_ref[...][:, None, :] + p_ref[...][None, :, :]

    out = pl.pallas_call(
        body,
        out_shape=jax.ShapeDtypeStruct((_VOCAB2, _NPOS, _EMBED), jnp.float32),
    )(W_word, W_pos)
    return out.reshape(_VOCAB2 * _NPOS, _EMBED)


def _sc_lookup(tok2d, pos2d, wcomb):
    mesh = plsc.VectorSubcoreMesh(
        core_axis_name="c", subcore_axis_name="s",
        num_cores=_NC, num_subcores=_NS)

    @functools.partial(
        pl.kernel,
        out_type=jax.ShapeDtypeStruct((_N, _EMBED), jnp.float32),
        mesh=mesh,
        scratch_types=[
            pltpu.VMEM((_IDXROWS_W, 128), jnp.int32),
            pltpu.VMEM((_IDXROWS_W, 128), jnp.int32),
            pltpu.VMEM((_IDXROWS_W, 128), jnp.int32),
            pltpu.VMEM((_CHUNK, _EMBED), jnp.float32),
            pltpu.SemaphoreType.DMA,
        ],
    )
    def k(tok_hbm, pos_hbm, comb_hbm, out_hbm, tok_v, pos_v, cidx_v, rows_v, gsem):
        c = lax.axis_index("c")
        s = lax.axis_index("s")
        wid = s * _NC + c
        ib = wid * _IDXROWS_W
        ob = wid * _ROWS_W

        pltpu.sync_copy(tok_hbm.at[pl.ds(ib, _IDXROWS_W)], tok_v)
        pltpu.sync_copy(pos_hbm.at[pl.ds(ib, _IDXROWS_W)], pos_v)

        def compute(j, carry):
            for i in range(8):
                sl = pl.ds(i * 16, 16)
                cidx_v[j, sl] = tok_v[j, sl] * _NPOS + pos_v[j, sl]
            return carry
        lax.fori_loop(0, _IDXROWS_W, compute, 0)

        def move(j, carry):
            pltpu.async_copy(comb_hbm.at[cidx_v.at[j]], rows_v, gsem).wait()
            pltpu.sync_copy(rows_v, out_hbm.at[pl.ds(ob + j * _CHUNK, _CHUNK)])
            return carry
        lax.fori_loop(0, _NCH, move, 0)

    return k(tok2d, pos2d, wcomb)


def kernel(tokens, pos, W_word, W_pos):
    wcomb = _build_comb(W_word, W_pos)
    tok2d = tokens.astype(jnp.int32).reshape(_N // 128, 128)
    pos2d = pos.astype(jnp.int32).reshape(_N // 128, 128)
    out = _sc_lookup(tok2d, pos2d, wcomb)
    return out.reshape(1024, 200, 128)


# speedup vs baseline: 10.0720x; 10.0720x over previous
import functools

import jax
import jax.numpy as jnp
from jax import lax
from jax.experimental import pallas as pl
from jax.experimental.pallas import tpu as pltpu
from jax.experimental.pallas import tpu_sc as plsc

_VOCAB2 = 1002
_NPOS = 24
_EMBED = 128
_NC, _NS = 2, 16
_NW = _NC * _NS
_N = 1024 * 200
_CHUNK = 128
_ROWS_W = _N // _NW
_NCH = _ROWS_W // _CHUNK
_IDXROWS_W = _ROWS_W // 128


def _build_comb(W_word, W_pos):
    def body(w_ref, p_ref, o_ref):
        o_ref[...] = w_ref[...][:, None, :] + p_ref[...][None, :, :]

    out = pl.pallas_call(
        body,
        out_shape=jax.ShapeDtypeStruct((_VOCAB2, _NPOS, _EMBED), jnp.float32),
    )(W_word, W_pos)
    return out.reshape(_VOCAB2 * _NPOS, _EMBED)


def _sc_lookup(tok2d, pos2d, wcomb):
    mesh = plsc.VectorSubcoreMesh(
        core_axis_name="c", subcore_axis_name="s",
        num_cores=_NC, num_subcores=_NS)

    @functools.partial(
        pl.kernel,
        out_type=jax.ShapeDtypeStruct((_N, _EMBED), jnp.float32),
        mesh=mesh,
        scratch_types=[
            pltpu.VMEM((_ROWS_W,), jnp.int32),
            pltpu.VMEM((_ROWS_W,), jnp.int32),
            pltpu.VMEM((_NCH, 128), jnp.int32),
            pltpu.VMEM((_CHUNK, _EMBED), jnp.float32),
            pltpu.SemaphoreType.DMA,
        ],
    )
    def k(tok_hbm, pos_hbm, comb_hbm, out_hbm, tok_v, pos_v, cidx_v, rows_v, gsem):
        c = lax.axis_index("c")
        s = lax.axis_index("s")
        wid = s * _NC + c
        ob = wid * _ROWS_W

        pltpu.sync_copy(tok_hbm.at[pl.ds(ob, _ROWS_W)], tok_v)
        pltpu.sync_copy(pos_hbm.at[pl.ds(ob, _ROWS_W)], pos_v)

        def compute(j, carry):
            for i in range(8):
                sl = pl.ds(j * 128 + i * 16, 16)
                cidx_v[j, pl.ds(i * 16, 16)] = tok_v[sl] * _NPOS + pos_v[sl]
            return carry
        lax.fori_loop(0, _NCH, compute, 0)

        def move(j, carry):
            pltpu.async_copy(comb_hbm.at[cidx_v.at[j]], rows_v, gsem).wait()
            pltpu.sync_copy(rows_v, out_hbm.at[pl.ds(ob + j * _CHUNK, _CHUNK)])
            return carry
        lax.fori_loop(0, _NCH, move, 0)

    return k(tok2d, pos2d, wcomb)


def kernel(tokens, pos, W_word, W_pos):
    wcomb = _build_comb(W_word, W_pos)
    tok_flat = tokens.astype(jnp.int32).reshape(_N)
    pos_flat = pos.astype(jnp.int32).reshape(_N)
    out = _sc_lookup(tok_flat, pos_flat, wcomb)
    return out.reshape(1024, 200, 128)


# speedup vs baseline: 13.2099x; 1.3116x over previous
import functools

import jax
import jax.numpy as jnp
from jax import lax
from jax.experimental import pallas as pl
from jax.experimental.pallas import tpu as pltpu
from jax.experimental.pallas import tpu_sc as plsc

_VOCAB2 = 1002
_NPOS = 24
_EMBED = 128
_NC, _NS = 2, 16
_NW = _NC * _NS
_N = 1024 * 200
_CHUNK = 128
_ROWS_W = _N // _NW
_NCH = _ROWS_W // _CHUNK
_NBUF = 5
_NGRP = _NCH // _NBUF


def _build_comb(W_word, W_pos):
    def body(w_ref, p_ref, o_ref):
        o_ref[...] = w_ref[...][:, None, :] + p_ref[...][None, :, :]

    out = pl.pallas_call(
        body,
        out_shape=jax.ShapeDtypeStruct((_VOCAB2, _NPOS, _EMBED), jnp.float32),
    )(W_word, W_pos)
    return out.reshape(_VOCAB2 * _NPOS, _EMBED)


def _sc_lookup(tok2d, pos2d, wcomb):
    mesh = plsc.VectorSubcoreMesh(
        core_axis_name="c", subcore_axis_name="s",
        num_cores=_NC, num_subcores=_NS)

    @functools.partial(
        pl.kernel,
        out_type=jax.ShapeDtypeStruct((_N, _EMBED), jnp.float32),
        mesh=mesh,
        scratch_types=[
            pltpu.VMEM((_ROWS_W,), jnp.int32),
            pltpu.VMEM((_ROWS_W,), jnp.int32),
            pltpu.VMEM((_NCH, 128), jnp.int32),
            [pltpu.VMEM((_CHUNK, _EMBED), jnp.float32) for _ in range(_NBUF)],
            [pltpu.SemaphoreType.DMA for _ in range(_NBUF)],
            [pltpu.SemaphoreType.DMA for _ in range(_NBUF)],
        ],
    )
    def k(tok_hbm, pos_hbm, comb_hbm, out_hbm, tok_v, pos_v, cidx_v,
          rows, gsem, ssem):
        c = lax.axis_index("c")
        s = lax.axis_index("s")
        wid = s * _NC + c
        ob = wid * _ROWS_W

        pltpu.sync_copy(tok_hbm.at[pl.ds(ob, _ROWS_W)], tok_v)
        pltpu.sync_copy(pos_hbm.at[pl.ds(ob, _ROWS_W)], pos_v)

        def compute(j, carry):
            for i in range(8):
                sl = pl.ds(j * 128 + i * 16, 16)
                cidx_v[j, pl.ds(i * 16, 16)] = tok_v[sl] * _NPOS + pos_v[sl]
            return carry
        lax.fori_loop(0, _NCH, compute, 0)

        def start_gather(b, j):
            pltpu.async_copy(comb_hbm.at[cidx_v.at[j]], rows[b], gsem[b])

        def wait_gather(b):
            pltpu.make_async_copy(comb_hbm.at[cidx_v.at[0]], rows[b],
                                  gsem[b]).wait()

        def start_scatter(b, j):
            pltpu.async_copy(rows[b], out_hbm.at[pl.ds(ob + j * _CHUNK, _CHUNK)],
                             ssem[b])

        def wait_scatter(b):
            pltpu.make_async_copy(rows[b], out_hbm.at[pl.ds(ob, _CHUNK)],
                                  ssem[b]).wait()

        for b in range(_NBUF):
            start_gather(b, b)

        def group(g, carry):
            base = g * _NBUF
            for b in range(_NBUF):
                wait_gather(b)
                start_scatter(b, base + b)

            @pl.when(g < _NGRP - 1)
            def _():
                for b in range(_NBUF):
                    wait_scatter(b)
                    start_gather(b, base + _NBUF + b)
            return carry
        lax.fori_loop(0, _NGRP, group, 0)

        for b in range(_NBUF):
            wait_scatter(b)

    return k(tok2d, pos2d, wcomb)


def kernel(tokens, pos, W_word, W_pos):
    wcomb = _build_comb(W_word, W_pos)
    tok_flat = tokens.astype(jnp.int32).reshape(_N)
    pos_flat = pos.astype(jnp.int32).reshape(_N)
    out = _sc_lookup(tok_flat, pos_flat, wcomb)
    return out.reshape(1024, 200, 128)


# speedup vs baseline: 13.5279x; 1.0241x over previous
import functools

import jax
import jax.numpy as jnp
from jax import lax
from jax.experimental import pallas as pl
from jax.experimental.pallas import tpu as pltpu
from jax.experimental.pallas import tpu_sc as plsc

_VOCAB2 = 1002
_NPOS = 24
_EMBED = 128
_NC, _NS = 2, 16
_NW = _NC * _NS
_N = 1024 * 200
_CHUNK = 64
_ROWS_W = _N // _NW
_NCH = _ROWS_W // _CHUNK
_NBUF = 10
_LOOK = 5
_NGRP = _NCH // _NBUF


def _build_comb(W_word, W_pos):
    def body(w_ref, p_ref, o_ref):
        o_ref[...] = w_ref[...][:, None, :] + p_ref[...][None, :, :]

    out = pl.pallas_call(
        body,
        out_shape=jax.ShapeDtypeStruct((_VOCAB2, _NPOS, _EMBED), jnp.float32),
    )(W_word, W_pos)
    return out.reshape(_VOCAB2 * _NPOS, _EMBED)


def _sc_lookup(tok2d, pos2d, wcomb):
    mesh = plsc.VectorSubcoreMesh(
        core_axis_name="c", subcore_axis_name="s",
        num_cores=_NC, num_subcores=_NS)

    @functools.partial(
        pl.kernel,
        out_type=jax.ShapeDtypeStruct((_N, _EMBED), jnp.float32),
        mesh=mesh,
        scratch_types=[
            pltpu.VMEM((_ROWS_W,), jnp.int32),
            pltpu.VMEM((_ROWS_W,), jnp.int32),
            pltpu.VMEM((_NCH, _CHUNK), jnp.int32),
            [pltpu.VMEM((_CHUNK, _EMBED), jnp.float32) for _ in range(_NBUF)],
            [pltpu.SemaphoreType.DMA for _ in range(_NBUF)],
            [pltpu.SemaphoreType.DMA for _ in range(_NBUF)],
        ],
    )
    def k(tok_hbm, pos_hbm, comb_hbm, out_hbm, tok_v, pos_v, cidx_v,
          rows, gsem, ssem):
        c = lax.axis_index("c")
        s = lax.axis_index("s")
        wid = s * _NC + c
        ob = wid * _ROWS_W

        pltpu.sync_copy(tok_hbm.at[pl.ds(ob, _ROWS_W)], tok_v)
        pltpu.sync_copy(pos_hbm.at[pl.ds(ob, _ROWS_W)], pos_v)

        def compute(j, carry):
            for i in range(_CHUNK // 16):
                sl = pl.ds(j * _CHUNK + i * 16, 16)
                cidx_v[j, pl.ds(i * 16, 16)] = tok_v[sl] * _NPOS + pos_v[sl]
            return carry
        lax.fori_loop(0, _NCH, compute, 0)

        def start_gather(b, j):
            pltpu.async_copy(comb_hbm.at[cidx_v.at[j]], rows[b], gsem[b])

        def wait_gather(b):
            pltpu.make_async_copy(comb_hbm.at[cidx_v.at[0]], rows[b],
                                  gsem[b]).wait()

        def start_scatter(b, j):
            pltpu.async_copy(rows[b], out_hbm.at[pl.ds(ob + j * _CHUNK, _CHUNK)],
                             ssem[b])

        def wait_scatter(b):
            pltpu.make_async_copy(rows[b], out_hbm.at[pl.ds(ob, _CHUNK)],
                                  ssem[b]).wait()

        for b in range(_LOOK):
            start_gather(b, b)

        def group(g, carry):
            base = g * _NBUF
            for b in range(_NBUF):
                j = base + b
                bg = (b + _LOOK) % _NBUF

                @pl.when(j + _LOOK < _NCH)
                def _():
                    @pl.when(j >= _NBUF - _LOOK)
                    def _():
                        wait_scatter(bg)
                    start_gather(bg, j + _LOOK)

                wait_gather(b)
                start_scatter(b, j)
            return carry
        lax.fori_loop(0, _NGRP, group, 0)

        for b in range(_NBUF):
            wait_scatter(b)

    return k(tok2d, pos2d, wcomb)


def kernel(tokens, pos, W_word, W_pos):
    wcomb = _build_comb(W_word, W_pos)
    tok_flat = tokens.astype(jnp.int32).reshape(_N)
    pos_flat = pos.astype(jnp.int32).reshape(_N)
    out = _sc_lookup(tok_flat, pos_flat, wcomb)
    return out.reshape(1024, 200, 128)


# speedup vs baseline: 13.9114x; 1.0284x over previous
import functools

import jax
import jax.numpy as jnp
from jax import lax
from jax.experimental import pallas as pl
from jax.experimental.pallas import tpu as pltpu
from jax.experimental.pallas import tpu_sc as plsc

_VOCAB2 = 1002
_NPOS = 24
_EMBED = 128
_NC, _NS = 2, 16
_NW = _NC * _NS
_BATCH = 1024
_HIST = 200
_N = _BATCH * _HIST
_BR_W = _BATCH // _NW
_NBUF = 4
_LOOK = 2
_NGRP = _BR_W // _NBUF
_SPLIT = 128
_REM = _HIST - _SPLIT


def _build_comb(W_word, W_pos):
    def body(w_ref, p_ref, o_ref):
        o_ref[...] = w_ref[...][:, None, :] + p_ref[...][None, :, :]

    out = pl.pallas_call(
        body,
        out_shape=jax.ShapeDtypeStruct((_VOCAB2, _NPOS, _EMBED), jnp.float32),
    )(W_word, W_pos)
    return out.reshape(_VOCAB2 * _NPOS, _EMBED)


def _sc_lookup(tokens, pos, wcomb):
    mesh = plsc.VectorSubcoreMesh(
        core_axis_name="c", subcore_axis_name="s",
        num_cores=_NC, num_subcores=_NS)

    @functools.partial(
        pl.kernel,
        out_type=jax.ShapeDtypeStruct((_N, _EMBED), jnp.float32),
        mesh=mesh,
        scratch_types=[
            pltpu.VMEM((_BR_W, _HIST), jnp.int32),
            pltpu.VMEM((_BR_W, _HIST), jnp.int32),
            pltpu.VMEM((_BR_W, _HIST), jnp.int32),
            [pltpu.VMEM((_HIST, _EMBED), jnp.float32) for _ in range(_NBUF)],
            [pltpu.SemaphoreType.DMA for _ in range(_NBUF)],
            [pltpu.SemaphoreType.DMA for _ in range(_NBUF)],
        ],
    )
    def k(tok_hbm, pos_hbm, comb_hbm, out_hbm, tok_v, pos_v, cidx_v,
          rows, gsem, ssem):
        c = lax.axis_index("c")
        s = lax.axis_index("s")
        wid = s * _NC + c
        rb = wid * _BR_W

        pltpu.sync_copy(tok_hbm.at[pl.ds(rb, _BR_W)], tok_v)
        pltpu.sync_copy(pos_hbm.at[pl.ds(rb, _BR_W)], pos_v)

        starts = list(range(0, _HIST - 16, 16)) + [_HIST - 16]

        def compute(r, carry):
            for st in starts:
                sl = pl.ds(st, 16)
                cidx_v[r, sl] = tok_v[r, sl] * _NPOS + pos_v[r, sl]
            return carry
        lax.fori_loop(0, _BR_W, compute, 0)

        def start_gather(b, r):
            pltpu.async_copy(comb_hbm.at[cidx_v.at[r, pl.ds(0, _SPLIT)]],
                             rows[b].at[pl.ds(0, _SPLIT)], gsem[b])
            pltpu.async_copy(comb_hbm.at[cidx_v.at[r, pl.ds(_SPLIT, _REM)]],
                             rows[b].at[pl.ds(_SPLIT, _REM)], gsem[b])

        def wait_gather(b):
            pltpu.make_async_copy(comb_hbm.at[cidx_v.at[0, pl.ds(0, _SPLIT)]],
                                  rows[b].at[pl.ds(0, _SPLIT)], gsem[b]).wait()
            pltpu.make_async_copy(comb_hbm.at[cidx_v.at[0, pl.ds(_SPLIT, _REM)]],
                                  rows[b].at[pl.ds(_SPLIT, _REM)], gsem[b]).wait()

        def start_scatter(b, r):
            pltpu.async_copy(rows[b],
                             out_hbm.at[pl.ds((rb + r) * _HIST, _HIST)], ssem[b])

        def wait_scatter(b):
            pltpu.make_async_copy(rows[b], out_hbm.at[pl.ds(0, _HIST)],
                                  ssem[b]).wait()

        for b in range(_LOOK):
            start_gather(b, b)

        def group(g, carry):
            base = g * _NBUF
            for b in range(_NBUF):
                r = base + b
                bg = (b + _LOOK) % _NBUF

                @pl.when(r + _LOOK < _BR_W)
                def _():
                    @pl.when(r >= _NBUF - _LOOK)
                    def _():
                        wait_scatter(bg)
                    start_gather(bg, r + _LOOK)

                wait_gather(b)
                start_scatter(b, r)
            return carry
        lax.fori_loop(0, _NGRP, group, 0)

        for b in range(_NBUF):
            wait_scatter(b)

    return k(tokens, pos, wcomb)


def kernel(tokens, pos, W_word, W_pos):
    wcomb = _build_comb(W_word, W_pos)
    out = _sc_lookup(tokens.astype(jnp.int32), pos.astype(jnp.int32), wcomb)
    return out.reshape(_BATCH, _HIST, _EMBED)


# speedup vs baseline: 13.9633x; 1.0037x over previous
import functools

import jax
import jax.numpy as jnp
from jax import lax
from jax.experimental import pallas as pl
from jax.experimental.pallas import tpu as pltpu
from jax.experimental.pallas import tpu_sc as plsc

_VOCAB2 = 1002
_NPOS = 24
_EMBED = 128
_NC, _NS = 2, 16
_NW = _NC * _NS
_BATCH = 1024
_HIST = 200
_N = _BATCH * _HIST
_BR_W = _BATCH // _NW
_NBUF = 4
_LOOK = 2
_NGRP = _BR_W // _NBUF
_SPLIT = 128
_REM = _HIST - _SPLIT


def _build_comb(W_word, W_pos, tokens, pos):
    def body(w_ref, p_ref, t_ref, q_ref, comb_ref, cidx_ref):
        comb_ref[...] = w_ref[...][:, None, :] + p_ref[...][None, :, :]
        cidx_ref[...] = t_ref[...] * _NPOS + q_ref[...]

    comb, cidx = pl.pallas_call(
        body,
        out_shape=[
            jax.ShapeDtypeStruct((_VOCAB2, _NPOS, _EMBED), jnp.float32),
            jax.ShapeDtypeStruct((_BATCH, _HIST), jnp.int32),
        ],
    )(W_word, W_pos, tokens, pos)
    return comb.reshape(_VOCAB2 * _NPOS, _EMBED), cidx


def _sc_lookup(cidx, wcomb):
    mesh = plsc.VectorSubcoreMesh(
        core_axis_name="c", subcore_axis_name="s",
        num_cores=_NC, num_subcores=_NS)

    @functools.partial(
        pl.kernel,
        out_type=jax.ShapeDtypeStruct((_N, _EMBED), jnp.float32),
        mesh=mesh,
        scratch_types=[
            pltpu.VMEM((_BR_W, _HIST), jnp.int32),
            [pltpu.VMEM((_HIST, _EMBED), jnp.float32) for _ in range(_NBUF)],
            [pltpu.SemaphoreType.DMA for _ in range(_NBUF)],
            [pltpu.SemaphoreType.DMA for _ in range(_NBUF)],
        ],
    )
    def k(cidx_hbm, comb_hbm, out_hbm, cidx_v, rows, gsem, ssem):
        c = lax.axis_index("c")
        s = lax.axis_index("s")
        wid = s * _NC + c
        rb = wid * _BR_W

        pltpu.sync_copy(cidx_hbm.at[pl.ds(rb, _BR_W)], cidx_v)

        def start_gather(b, r):
            pltpu.async_copy(comb_hbm.at[cidx_v.at[r, pl.ds(0, _SPLIT)]],
                             rows[b].at[pl.ds(0, _SPLIT)], gsem[b])
            pltpu.async_copy(comb_hbm.at[cidx_v.at[r, pl.ds(_SPLIT, _REM)]],
                             rows[b].at[pl.ds(_SPLIT, _REM)], gsem[b])

        def wait_gather(b):
            pltpu.make_async_copy(comb_hbm.at[cidx_v.at[0, pl.ds(0, _SPLIT)]],
                                  rows[b].at[pl.ds(0, _SPLIT)], gsem[b]).wait()
            pltpu.make_async_copy(comb_hbm.at[cidx_v.at[0, pl.ds(_SPLIT, _REM)]],
                                  rows[b].at[pl.ds(_SPLIT, _REM)], gsem[b]).wait()

        def start_scatter(b, r):
            pltpu.async_copy(rows[b],
                             out_hbm.at[pl.ds((rb + r) * _HIST, _HIST)], ssem[b])

        def wait_scatter(b):
            pltpu.make_async_copy(rows[b], out_hbm.at[pl.ds(0, _HIST)],
                                  ssem[b]).wait()

        for b in range(_LOOK):
            start_gather(b, b)

        def group(g, carry):
            base = g * _NBUF
            for b in range(_NBUF):
                r = base + b
                bg = (b + _LOOK) % _NBUF

                @pl.when(r + _LOOK < _BR_W)
                def _():
                    @pl.when(r >= _NBUF - _LOOK)
                    def _():
                        wait_scatter(bg)
                    start_gather(bg, r + _LOOK)

                wait_gather(b)
                start_scatter(b, r)
            return carry
        lax.fori_loop(0, _NGRP, group, 0)

        for b in range(_NBUF):
            wait_scatter(b)

    return k(cidx, wcomb)


def kernel(tokens, pos, W_word, W_pos):
    wcomb, cidx = _build_comb(W_word, W_pos,
                              tokens.astype(jnp.int32), pos.astype(jnp.int32))
    out = _sc_lookup(cidx, wcomb)
    return out.reshape(_BATCH, _HIST, _EMBED)
